# baseline (speedup 1.0000x reference)
import functools

import jax
import jax.numpy as jnp
from jax import lax
from jax.experimental import pallas as pl
from jax.experimental.pallas import tpu as pltpu
from jax.experimental.pallas import tpu_sc as plsc

_NC = 2
_NS = 16
_NW = _NC * _NS
_CHUNK = 64
_NBUF = 12
_LAG = 6


@functools.partial(jax.jit, static_argnums=(2, 3, 4))
def _sc_gather(idx3, table, nchunks, chunk, dim):
    b_per_w = nchunks * chunk
    total = _NW * b_per_w
    nsup = -(-(nchunks + _LAG) // _NBUF)

    mesh = plsc.VectorSubcoreMesh(
        core_axis_name="c", subcore_axis_name="s",
        num_cores=_NC, num_subcores=_NS,
    )

    def body(idx_hbm, table_hbm, out_hbm, idx_v, rows_v, *sems):
        gsem = sems[:_NBUF]
        osem = sems[_NBUF:]
        wid = lax.axis_index("s") * _NC + lax.axis_index("c")
        pltpu.sync_copy(idx_hbm.at[wid], idx_v)
        base_w = wid * b_per_w

        def fire_gather(c, b):
            pltpu.async_copy(table_hbm.at[idx_v.at[c]], rows_v.at[b], gsem[b])

        def wait_gather(b):
            pltpu.make_async_copy(
                table_hbm.at[pl.ds(0, chunk)], rows_v.at[b], gsem[b]).wait()

        def fire_out(c, b):
            pltpu.async_copy(
                rows_v.at[b], out_hbm.at[pl.ds(base_w + c * chunk, chunk)],
                osem[b])

        def wait_out(b):
            pltpu.make_async_copy(
                table_hbm.at[pl.ds(0, chunk)], rows_v.at[b], osem[b]).wait()

        def super_step(i, carry):
            for b in range(_NBUF):
                c = i * _NBUF + b

                @pl.when(jnp.logical_and(c >= _NBUF, c < nchunks))
                def _():
                    wait_out(b)

                @pl.when(c < nchunks)
                def _():
                    fire_gather(c, b)

                c2 = c - _LAG
                b2 = (b - _LAG) % _NBUF

                @pl.when(jnp.logical_and(c2 >= 0, c2 < nchunks))
                def _():
                    wait_gather(b2)
                    fire_out(c2, b2)
            return carry

        lax.fori_loop(0, nsup, super_step, 0)

        for b in range(_NBUF):
            wait_out(b)

    scratch = [
        pltpu.VMEM((nchunks, chunk), jnp.int32),
        pltpu.VMEM((_NBUF, chunk, dim), jnp.float32),
    ] + [pltpu.SemaphoreType.DMA] * (2 * _NBUF)

    return pl.kernel(
        body,
        out_type=jax.ShapeDtypeStruct((total, dim), jnp.float32),
        mesh=mesh,
        scratch_types=scratch,
    )(idx3, table)


def kernel(input, table):
    batch, hist = input.shape
    vocab, dim = table.shape
    total = batch * hist
    assert total % _NW == 0
    b_per_w = total // _NW
    assert b_per_w % _CHUNK == 0
    nchunks = b_per_w // _CHUNK
    assert nchunks >= _NBUF
    idx3 = input.reshape(_NW, nchunks, _CHUNK)
    out = _sc_gather(idx3, table, nchunks, _CHUNK, dim)
    return out.reshape(batch, hist, dim)


# speedup vs baseline: 1.0033x; 1.0033x over previous
import functools

import jax
import jax.numpy as jnp
from jax import lax
from jax.experimental import pallas as pl
from jax.experimental.pallas import tpu as pltpu
from jax.experimental.pallas import tpu_sc as plsc

_NC = 2
_NS = 16
_NW = _NC * _NS
_CHUNK = 128
_NBUF = 7
_LAG = 2


@functools.partial(jax.jit, static_argnums=(2, 3, 4))
def _sc_gather(idx3, table, nchunks, chunk, dim):
    b_per_w = nchunks * chunk
    total = _NW * b_per_w
    nsup = -(-(nchunks + _LAG) // _NBUF)

    mesh = plsc.VectorSubcoreMesh(
        core_axis_name="c", subcore_axis_name="s",
        num_cores=_NC, num_subcores=_NS,
    )

    def body(idx_hbm, table_hbm, out_hbm, idx_v, rows_v, *sems):
        gsem = sems[:_NBUF]
        osem = sems[_NBUF:]
        wid = lax.axis_index("s") * _NC + lax.axis_index("c")
        pltpu.sync_copy(idx_hbm.at[wid], idx_v)
        base_w = wid * b_per_w

        def fire_gather(c, b):
            pltpu.async_copy(table_hbm.at[idx_v.at[c]], rows_v.at[b], gsem[b])

        def wait_gather(b):
            pltpu.make_async_copy(
                table_hbm.at[pl.ds(0, chunk)], rows_v.at[b], gsem[b]).wait()

        def fire_out(c, b):
            pltpu.async_copy(
                rows_v.at[b], out_hbm.at[pl.ds(base_w + c * chunk, chunk)],
                osem[b])

        def wait_out(b):
            pltpu.make_async_copy(
                table_hbm.at[pl.ds(0, chunk)], rows_v.at[b], osem[b]).wait()

        def super_step(i, carry):
            for b in range(_NBUF):
                c = i * _NBUF + b

                @pl.when(jnp.logical_and(c >= _NBUF, c < nchunks))
                def _():
                    wait_out(b)

                @pl.when(c < nchunks)
                def _():
                    fire_gather(c, b)

                c2 = c - _LAG
                b2 = (b - _LAG) % _NBUF

                @pl.when(jnp.logical_and(c2 >= 0, c2 < nchunks))
                def _():
                    wait_gather(b2)
                    fire_out(c2, b2)
            return carry

        lax.fori_loop(0, nsup, super_step, 0)

        for b in range(_NBUF):
            wait_out(b)

    scratch = [
        pltpu.VMEM((nchunks, chunk), jnp.int32),
        pltpu.VMEM((_NBUF, chunk, dim), jnp.float32),
    ] + [pltpu.SemaphoreType.DMA] * (2 * _NBUF)

    return pl.kernel(
        body,
        out_type=jax.ShapeDtypeStruct((total, dim), jnp.float32),
        mesh=mesh,
        scratch_types=scratch,
    )(idx3, table)


def kernel(input, table):
    batch, hist = input.shape
    vocab, dim = table.shape
    total = batch * hist
    assert total % _NW == 0
    b_per_w = total // _NW
    assert b_per_w % _CHUNK == 0
    nchunks = b_per_w // _CHUNK
    assert nchunks >= _NBUF
    idx3 = input.reshape(_NW, nchunks, _CHUNK)
    out = _sc_gather(idx3, table, nchunks, _CHUNK, dim)
    return out.reshape(batch, hist, dim)


# speedup vs baseline: 1.0052x; 1.0019x over previous
import functools

import jax
import jax.numpy as jnp
from jax import lax
from jax.experimental import pallas as pl
from jax.experimental.pallas import tpu as pltpu
from jax.experimental.pallas import tpu_sc as plsc

_NC = 2
_NS = 16
_NW = _NC * _NS
_CHUNK = 128
_NBUF = 7
_LAG = 4


@functools.partial(jax.jit, static_argnums=(2, 3, 4))
def _sc_gather(idx3, table, nchunks, chunk, dim):
    b_per_w = nchunks * chunk
    total = _NW * b_per_w
    nsup = -(-(nchunks + _LAG) // _NBUF)

    mesh = plsc.VectorSubcoreMesh(
        core_axis_name="c", subcore_axis_name="s",
        num_cores=_NC, num_subcores=_NS,
    )

    def body(idx_hbm, table_hbm, out_hbm, idx_v, rows_v, *sems):
        gsem = sems[:_NBUF]
        osem = sems[_NBUF:]
        wid = lax.axis_index("s") * _NC + lax.axis_index("c")
        pltpu.sync_copy(idx_hbm.at[wid], idx_v)
        base_w = wid * b_per_w

        def fire_gather(c, b):
            pltpu.async_copy(table_hbm.at[idx_v.at[c]], rows_v.at[b], gsem[b])

        def wait_gather(b):
            pltpu.make_async_copy(
                table_hbm.at[pl.ds(0, chunk)], rows_v.at[b], gsem[b]).wait()

        def fire_out(c, b):
            pltpu.async_copy(
                rows_v.at[b], out_hbm.at[pl.ds(base_w + c * chunk, chunk)],
                osem[b])

        def wait_out(b):
            pltpu.make_async_copy(
                table_hbm.at[pl.ds(0, chunk)], rows_v.at[b], osem[b]).wait()

        def super_step(i, carry):
            for b in range(_NBUF):
                c = i * _NBUF + b

                @pl.when(jnp.logical_and(c >= _NBUF, c < nchunks))
                def _():
                    wait_out(b)

                @pl.when(c < nchunks)
                def _():
                    fire_gather(c, b)

                c2 = c - _LAG
                b2 = (b - _LAG) % _NBUF

                @pl.when(jnp.logical_and(c2 >= 0, c2 < nchunks))
                def _():
                    wait_gather(b2)
                    fire_out(c2, b2)
            return carry

        lax.fori_loop(0, nsup, super_step, 0)

        for b in range(_NBUF):
            wait_out(b)

    scratch = [
        pltpu.VMEM((nchunks, chunk), jnp.int32),
        pltpu.VMEM((_NBUF, chunk, dim), jnp.float32),
    ] + [pltpu.SemaphoreType.DMA] * (2 * _NBUF)

    return pl.kernel(
        body,
        out_type=jax.ShapeDtypeStruct((total, dim), jnp.float32),
        mesh=mesh,
        scratch_types=scratch,
    )(idx3, table)


def kernel(input, table):
    batch, hist = input.shape
    vocab, dim = table.shape
    total = batch * hist
    assert total % _NW == 0
    b_per_w = total // _NW
    assert b_per_w % _CHUNK == 0
    nchunks = b_per_w // _CHUNK
    assert nchunks >= _NBUF
    idx3 = input.reshape(_NW, nchunks, _CHUNK)
    out = _sc_gather(idx3, table, nchunks, _CHUNK, dim)
    return out.reshape(batch, hist, dim)
